# baseline (speedup 1.0000x reference)
import functools

import jax
import jax.numpy as jnp
from jax import lax
from jax.experimental import pallas as pl
from jax.experimental.pallas import tpu as pltpu
from jax.experimental.pallas import tpu_sc as plsc

_NUM_ANSWERS = 16384
_D = 16


@functools.lru_cache(maxsize=None)
def _build_sc_kernel():
    info = plsc.get_sparse_core_info()
    nc, ns = info.num_cores, info.num_subcores
    nw = nc * ns
    bpw = _NUM_ANSWERS // nw
    rows_pw = 3 * bpw
    mesh = plsc.VectorSubcoreMesh(core_axis_name="c", subcore_axis_name="s")

    @functools.partial(
        pl.kernel,
        mesh=mesh,
        compiler_params=pltpu.CompilerParams(
            needs_layout_passes=False, use_tc_tiling_on_sc=False),
        out_type=(
            jax.ShapeDtypeStruct((_NUM_ANSWERS,), jnp.float32),
            jax.ShapeDtypeStruct((_NUM_ANSWERS,), jnp.float32),
        ),
        scratch_types=[
            pltpu.VMEM((rows_pw,), jnp.int32),
            pltpu.VMEM((rows_pw, _D), jnp.float32),
            pltpu.VMEM((bpw,), jnp.float32),
            pltpu.VMEM((bpw,), jnp.float32),
            pltpu.SemaphoreType.DMA,
        ],
    )
    def sc_kernel(idx_hbm, table_hbm, win_hbm, lose_hbm,
                  idx_v, rows_v, win_v, lose_v, sem):
        wid = lax.axis_index("s") * nc + lax.axis_index("c")
        base = wid * bpw
        pltpu.sync_copy(idx_hbm.at[pl.ds(base * 3, rows_pw)], idx_v)
        pltpu.async_copy(table_hbm.at[idx_v], rows_v, sem).wait()

        last_lane = lax.iota(jnp.int32, 16) == 15

        def body(i, carry):
            h = rows_v[3 * i, :]
            dw = h - rows_v[3 * i + 1, :]
            dl = h - rows_v[3 * i + 2, :]
            iv = jnp.full((16,), i, jnp.int32)
            sw = plsc.cumsum(dw * dw)
            sl = plsc.cumsum(dl * dl)
            plsc.store_scatter(win_v, [iv], sw, mask=last_lane)
            plsc.store_scatter(lose_v, [iv], sl, mask=last_lane)
            return carry

        lax.fori_loop(0, bpw, body, 0)
        pltpu.sync_copy(win_v, win_hbm.at[pl.ds(base, bpw)])
        pltpu.sync_copy(lose_v, lose_hbm.at[pl.ds(base, bpw)])

    return sc_kernel


def kernel(h_w_l, embedding):
    idx_flat = h_w_l.reshape(-1)
    win, lose = _build_sc_kernel()(idx_flat, embedding)
    return (win, lose)
